# baseline (speedup 1.0000x reference)
import functools

import jax
import jax.numpy as jnp
from jax import lax
from jax.experimental import pallas as pl
from jax.experimental.pallas import tpu as pltpu
from jax.experimental.pallas import tpu_sc as plsc

EMBED_DIM = 64

NC = 2
NS = 16
NW = NC * NS

G = 128
NBUF = 5


@functools.lru_cache(maxsize=None)
def _make_gather(B):
    b_per_w = B // NW
    ngrp = b_per_w // G
    nouter = ngrp // NBUF
    mesh = plsc.VectorSubcoreMesh(core_axis_name="c", subcore_axis_name="s")

    @functools.partial(
        pl.kernel,
        mesh=mesh,
        out_type=jax.ShapeDtypeStruct((B, EMBED_DIM), jnp.float32),
        compiler_params=pltpu.CompilerParams(use_tc_tiling_on_sc=False),
        scratch_types=(
            [
                pltpu.VMEM((ngrp, G), jnp.int32),
                pltpu.VMEM((NBUF, G, 128), jnp.float32),
            ]
            + [pltpu.SemaphoreType.DMA] * NBUF
        ),
    )
    def gather_kernel(idx_hbm, w_hbm, out_hbm, idx_v, rows_v, *sems):
        wid = lax.axis_index("s") * NC + lax.axis_index("c")
        base = wid * b_per_w

        pltpu.sync_copy(idx_hbm.at[wid], idx_v)

        def start(g, b):
            pltpu.make_async_copy(
                w_hbm.at[idx_v.at[g]], rows_v.at[b], sems[b]
            ).start()

        def drain(g, b):
            pltpu.make_async_copy(
                w_hbm.at[idx_v.at[g]], rows_v.at[b], sems[b]
            ).wait()
            pltpu.sync_copy(
                rows_v.at[b, :, pl.ds(0, EMBED_DIM)],
                out_hbm.at[pl.ds(base + g * G, G)],
            )

        for b in range(NBUF):
            start(b, b)

        def outer(o, carry):
            for b in range(NBUF):
                g = o * NBUF + b
                drain(g, b)
                start(g + NBUF, b)
            return carry

        lax.fori_loop(0, nouter - 1, outer, 0)

        for b in range(NBUF):
            drain((nouter - 1) * NBUF + b, b)

    return gather_kernel


def kernel(input_, weight):
    n, s = input_.shape
    B = n * s
    idx = input_.astype(jnp.int32).reshape(NW, B // (NW * G), G)
    w_pad = jnp.pad(weight, ((0, 0), (0, 128 - EMBED_DIM)))
    out = _make_gather(B)(idx, w_pad)
    o2 = jnp.transpose(out.reshape(n, s * EMBED_DIM))

    return jnp.transpose(o2.reshape(s, EMBED_DIM, n), (2, 0, 1))


# speedup vs baseline: 1.1100x; 1.1100x over previous
import functools

import jax
import jax.numpy as jnp
from jax import lax
from jax.experimental import pallas as pl
from jax.experimental.pallas import tpu as pltpu
from jax.experimental.pallas import tpu_sc as plsc

EMBED_DIM = 64

NC = 2
NS = 16
NW = NC * NS

G = 128
NSLOT = 2


@functools.lru_cache(maxsize=None)
def _make_gather(n, s):
    nsp = s // 2
    nbb = n // G
    bb_per_w = nbb // NW
    ng = nsp * bb_per_w
    mesh = plsc.VectorSubcoreMesh(core_axis_name="c", subcore_axis_name="s")

    @functools.partial(
        pl.kernel,
        mesh=mesh,
        out_type=jax.ShapeDtypeStruct((n // 8, nsp, 8, 128), jnp.float32),
        compiler_params=pltpu.CompilerParams(use_tc_tiling_on_sc=False),
        scratch_types=(
            [
                pltpu.VMEM((NSLOT, 1, 2, 1, G), jnp.int32),
                pltpu.VMEM((NSLOT, 2, G, 128), jnp.float32),
            ]
            + [pltpu.SemaphoreType.DMA] * NSLOT
            + [pltpu.SemaphoreType.DMA] * NSLOT
        ),
    )
    def gather_kernel(idx_hbm, w_hbm, out_hbm, idx_v, rows_v, *sems):
        gsem = sems[:NSLOT]
        isem = sems[NSLOT:]
        wid = lax.axis_index("s") * NC + lax.axis_index("c")

        def split(g):
            return g >> 2, wid * bb_per_w + (g & (bb_per_w - 1))

        def idx_copy(g, t, sem):
            sp, bb = split(g)
            return pltpu.make_async_copy(
                idx_hbm.at[pl.ds(sp, 1), :, pl.ds(bb, 1), :],
                idx_v.at[t], sem,
            )

        def gather_copy(g, t, par):
            return pltpu.make_async_copy(
                w_hbm.at[idx_v.at[t, 0, par, 0]],
                rows_v.at[t, par], gsem[t],
            )

        def start_gathers(g, t):
            idx_copy(g, t, isem[t]).wait()
            gather_copy(g, t, 0).start()
            gather_copy(g, t, 1).start()

        def drain(g, t):
            sp, bb = split(g)
            gather_copy(g, t, 0).wait()
            gather_copy(g, t, 1).wait()
            for i in range(G // 8):
                for par in range(2):
                    pltpu.sync_copy(
                        rows_v.at[t, par, pl.ds(8 * i, 8), pl.ds(0, EMBED_DIM)],
                        out_hbm.at[
                            bb * (G // 8) + i, sp, :,
                            pl.ds(par * EMBED_DIM, EMBED_DIM),
                        ],
                    )

        idx_copy(0, 0, isem[0]).start()
        idx_copy(1, 1, isem[1]).start()
        start_gathers(0, 0)

        def step(g, t):
            u = 1 - t
            start_gathers(g + 1, u)
            drain(g, t)
            idx_copy(g + 2, t, isem[t]).start()

        def body2(o, carry):
            g = o * 2
            step(g, 0)
            step(g + 1, 1)
            return carry

        lax.fori_loop(0, (ng - 2) // 2, body2, 0)

        start_gathers(ng - 1, (ng - 1) % 2)
        drain(ng - 2, (ng - 2) % 2)
        drain(ng - 1, (ng - 1) % 2)

    return gather_kernel


def kernel(input_, weight):
    n, s = input_.shape
    idx4 = input_.astype(jnp.int32).T.reshape(s // 2, 2, n // G, G)
    w_pad = jnp.pad(weight, ((0, 0), (0, 128 - EMBED_DIM)))
    out4 = _make_gather(n, s)(idx4, w_pad)
    m = out4.transpose(0, 2, 1, 3).reshape(n, (s // 2) * 128)
    o2 = jnp.transpose(m)
    return jnp.transpose(o2.reshape(s, EMBED_DIM, n), (2, 0, 1))
